# baseline (speedup 1.0000x reference)
import functools

import jax
import jax.numpy as jnp
from jax import lax
from jax.experimental import pallas as pl
from jax.experimental.pallas import tpu as pltpu
from jax.experimental.pallas import tpu_sc as plsc

ALPHA = 4


def _make_sc_gather(n_rows_out, per_w, T, S, D, NC, dtype):
    mesh = plsc.VectorSubcoreMesh(core_axis_name="c", subcore_axis_name="s")

    @functools.partial(
        pl.kernel,
        out_type=jax.ShapeDtypeStruct((n_rows_out, D), dtype),
        mesh=mesh,
        scratch_types=[
            pltpu.VMEM((2, D), dtype),
            pltpu.SemaphoreType.DMA((2,)),
        ],
    )
    def sc_gather(x_hbm, out_hbm, buf, sem):
        wid = lax.axis_index("s") * NC + lax.axis_index("c")
        base = wid * per_w

        def in_row(r):
            bc = r // S
            s = r % S
            return bc * T + (T - 1) * s // (S - 1)

        def start(j):
            r = base + j
            pltpu.make_async_copy(
                x_hbm.at[in_row(r)], buf.at[j % 2], sem.at[j % 2]
            ).start()

        start(0)
        for j in range(per_w):
            if j + 1 < per_w:
                start(j + 1)
            r = base + j
            pltpu.make_async_copy(
                x_hbm.at[in_row(r)], buf.at[j % 2], sem.at[j % 2]
            ).wait()
            pltpu.sync_copy(buf.at[j % 2], out_hbm.at[r])

    return sc_gather


def kernel(frames):
    B, C, T, H, W = frames.shape
    S = T // ALPHA
    D = H * W
    info = plsc.get_sparse_core_info()
    NC, NS = info.num_cores, info.num_subcores
    NW = NC * NS
    n_out = B * C * S
    per_w = n_out // NW
    x = frames.reshape(B * C * T, D)
    slow = _make_sc_gather(n_out, per_w, T, S, D, NC, frames.dtype)(x)
    return slow.reshape(B, C, S, H, W), jnp.zeros((8,), frames.dtype)


# speedup vs baseline: 1.7712x; 1.7712x over previous
import jax
import jax.numpy as jnp
import numpy as np
from jax.experimental import pallas as pl
from jax.experimental.pallas import tpu as pltpu

ALPHA = 4


def _copy_body(in_ref, out_ref):
    out_ref[...] = in_ref[...]


def kernel(frames):
    B, C, T, H, W = frames.shape
    S = T // ALPHA
    slow = pl.pallas_call(
        _copy_body,
        grid=(S,),
        in_specs=[
            pl.BlockSpec(
                (B, C, 1, H, W), lambda t: (0, 0, (T - 1) * t // (S - 1), 0, 0)
            )
        ],
        out_specs=pl.BlockSpec((B, C, 1, H, W), lambda t: (0, 0, t, 0, 0)),
        out_shape=jax.ShapeDtypeStruct((B, C, S, H, W), frames.dtype),
    )(frames)
    return slow, frames


# speedup vs baseline: 1.9524x; 1.1023x over previous
import jax
import jax.numpy as jnp
import numpy as np
from jax.experimental import pallas as pl
from jax.experimental.pallas import tpu as pltpu

ALPHA = 4


def _make_body(idx):
    def body(in_ref, slow_ref, fast_ref):
        fast_ref[...] = in_ref[...]
        for s, i in enumerate(idx):
            slow_ref[0, 0, s] = in_ref[0, 0, i]

    return body


def kernel(frames):
    B, C, T, H, W = frames.shape
    S = T // ALPHA
    idx = [int(v) for v in np.linspace(0.0, T - 1, S, dtype=np.float32).astype(np.int32)]
    slow, fast = pl.pallas_call(
        _make_body(idx),
        grid=(B, C),
        in_specs=[pl.BlockSpec((1, 1, T, H, W), lambda b, c: (b, c, 0, 0, 0))],
        out_specs=[
            pl.BlockSpec((1, 1, S, H, W), lambda b, c: (b, c, 0, 0, 0)),
            pl.BlockSpec((1, 1, T, H, W), lambda b, c: (b, c, 0, 0, 0)),
        ],
        out_shape=[
            jax.ShapeDtypeStruct((B, C, S, H, W), frames.dtype),
            jax.ShapeDtypeStruct((B, C, T, H, W), frames.dtype),
        ],
    )(frames)
    return slow, fast
